# baseline (speedup 1.0000x reference)
import jax
import jax.numpy as jnp
from jax.experimental import pallas as pl
from jax.experimental.pallas import tpu as pltpu

N = 10000
D = 64
ALPHA = 0.5
ROWS_BLK = 400


def _xw_body(x_ref, w_ref, o_ref):
    o_ref[...] = jnp.dot(x_ref[...], w_ref[...], preferred_element_type=jnp.float32)


def _xw(x, w):
    return pl.pallas_call(
        _xw_body,
        out_shape=jax.ShapeDtypeStruct((x.shape[0], w.shape[1]), jnp.float32),
    )(x, w)


def _gcn_body(a_ref, s_ref, init_ref, o_ref):
    y = jnp.dot(a_ref[...], s_ref[...], preferred_element_type=jnp.float32)
    o_ref[...] = jnp.maximum(ALPHA * y + (1.0 - ALPHA) * init_ref[...], 0.0)


def _gcn_layer(adj, s, init):
    return pl.pallas_call(
        _gcn_body,
        grid=(N // ROWS_BLK,),
        in_specs=[
            pl.BlockSpec((ROWS_BLK, N), lambda i: (i, 0)),
            pl.BlockSpec((N, D), lambda i: (0, 0)),
            pl.BlockSpec((ROWS_BLK, D), lambda i: (i, 0)),
        ],
        out_specs=pl.BlockSpec((ROWS_BLK, D), lambda i: (i, 0)),
        out_shape=jax.ShapeDtypeStruct((N, D), jnp.float32),
    )(adj, s, init)


def _rnn_body(ce_ref, td_ref, ft_ref, p_ref, whh_ref, wf_ref, wt_ref, wp_ref,
              b_ref, o_ref):
    const = jnp.dot(p_ref[...], wp_ref[...], preferred_element_type=jnp.float32)
    const = const + b_ref[...]
    z = jnp.dot(ce_ref[...], whh_ref[...], preferred_element_type=jnp.float32)
    z = z + jnp.dot(ft_ref[...], wf_ref[...], preferred_element_type=jnp.float32)
    z = z + td_ref[...] * wt_ref[...]
    h = jnp.tanh(z + const)
    nrm = jnp.sqrt(jnp.sum(h * h, axis=1, keepdims=True))
    o_ref[...] = h / jnp.maximum(nrm, 1e-12)


def _rnn(code_embed, timediffs, features, patient_row, W_hh_T, W_f_T, w_t_row,
         W_p_T, b):
    B = code_embed.shape[0]
    return pl.pallas_call(
        _rnn_body,
        out_shape=jax.ShapeDtypeStruct((B, D), jnp.float32),
    )(code_embed, timediffs, features, patient_row, W_hh_T, W_f_T, w_t_row,
      W_p_T, b)


def kernel(patient_dynamic, code_dynamic, init_code_dynamic, adj, patientid,
           codeid, ancestorid, features, timediffs, W1, W2, W_ih, b_ih, W_hh,
           b_hh):
    s1 = _xw(code_dynamic, W1)
    x1 = _gcn_layer(adj, s1, init_code_dynamic)
    s2 = _xw(x1, W2)
    x2 = _gcn_layer(adj, s2, init_code_dynamic)
    code_embed = jnp.take(x2, codeid, axis=0)

    patient_row = jax.lax.dynamic_slice_in_dim(patient_dynamic, patientid, 1,
                                               axis=0)
    W_p_T = W_ih[:, :D].T
    w_t_row = W_ih[:, D:D + 1].T
    W_f_T = W_ih[:, D + 1:].T
    b = (b_ih + b_hh)[None, :]
    return _rnn(code_embed, timediffs, features, patient_row, W_hh.T, W_f_T,
                w_t_row, W_p_T, b)


# speedup vs baseline: 1.3902x; 1.3902x over previous
import jax
import jax.numpy as jnp
from jax.experimental import pallas as pl
from jax.experimental.pallas import tpu as pltpu

N = 10000
D = 64
ALPHA = 0.5
ROWS_BLK = 400
GB = 256


def _xw_body(x_ref, w_ref, o_ref):
    o_ref[...] = jnp.dot(x_ref[...], w_ref[...], preferred_element_type=jnp.float32)


def _xw(x, w):
    return pl.pallas_call(
        _xw_body,
        out_shape=jax.ShapeDtypeStruct((x.shape[0], w.shape[1]), jnp.float32),
    )(x, w)


def _gcn_body(a_ref, s_ref, init_ref, o_ref):
    y = jnp.dot(a_ref[...], s_ref[...], preferred_element_type=jnp.float32)
    o_ref[...] = jnp.maximum(ALPHA * y + (1.0 - ALPHA) * init_ref[...], 0.0)


def _gcn_layer(adj, s, init):
    return pl.pallas_call(
        _gcn_body,
        grid=(N // ROWS_BLK,),
        in_specs=[
            pl.BlockSpec((ROWS_BLK, N), lambda i: (i, 0)),
            pl.BlockSpec((N, D), lambda i: (0, 0)),
            pl.BlockSpec((ROWS_BLK, D), lambda i: (i, 0)),
        ],
        out_specs=pl.BlockSpec((ROWS_BLK, D), lambda i: (i, 0)),
        out_shape=jax.ShapeDtypeStruct((N, D), jnp.float32),
    )(adj, s, init)


def _l2_body(id_ref, adj_ref, init_ref, s_ref, td_ref, ft_ref, p_ref, whh_ref,
             wf_ref, wt_ref, wp_ref, b_ref, o_ref, a_buf, i_buf, sem, isem):
    i = pl.program_id(0)
    nblk = pl.num_programs(0)

    def issue(blk, slot):
        def body(j, _):
            r = id_ref[blk * GB + j]
            pltpu.make_async_copy(
                adj_ref.at[pl.ds(r, 1), :], a_buf.at[slot, pl.ds(j, 1), :],
                sem.at[slot]).start()
            pltpu.make_async_copy(
                init_ref.at[pl.ds(r, 1), :], i_buf.at[slot, pl.ds(j, 1), :],
                isem.at[slot]).start()
            return 0
        jax.lax.fori_loop(0, GB, body, 0)

    def wait(slot):
        def body(j, _):
            pltpu.make_async_copy(
                adj_ref.at[pl.ds(0, 1), :], a_buf.at[slot, pl.ds(0, 1), :],
                sem.at[slot]).wait()
            pltpu.make_async_copy(
                init_ref.at[pl.ds(0, 1), :], i_buf.at[slot, pl.ds(0, 1), :],
                isem.at[slot]).wait()
            return 0
        jax.lax.fori_loop(0, GB, body, 0)

    @pl.when(i == 0)
    def _():
        issue(0, 0)

    @pl.when(i + 1 < nblk)
    def _():
        issue(i + 1, (i + 1) % 2)

    slot = i % 2
    wait(slot)
    y = jnp.dot(a_buf[slot], s_ref[...], preferred_element_type=jnp.float32)
    x2g = jnp.maximum(ALPHA * y + (1.0 - ALPHA) * i_buf[slot], 0.0)
    const = jnp.dot(p_ref[...], wp_ref[...], preferred_element_type=jnp.float32)
    z = jnp.dot(x2g, whh_ref[...], preferred_element_type=jnp.float32)
    z = z + jnp.dot(ft_ref[...], wf_ref[...], preferred_element_type=jnp.float32)
    z = z + td_ref[...] * wt_ref[...] + const + b_ref[...]
    h = jnp.tanh(z)
    nrm = jnp.sqrt(jnp.sum(h * h, axis=1, keepdims=True))
    o_ref[...] = h / jnp.maximum(nrm, 1e-12)


def _layer2_fused(codeid, adj, init, s2, timediffs, features, patient_row,
                  W_hh_T, W_f_T, w_t_row, W_p_T, b):
    B = codeid.shape[0]
    grid_spec = pltpu.PrefetchScalarGridSpec(
        num_scalar_prefetch=1,
        grid=(B // GB,),
        in_specs=[
            pl.BlockSpec(memory_space=pltpu.MemorySpace.HBM),
            pl.BlockSpec(memory_space=pltpu.MemorySpace.HBM),
            pl.BlockSpec((N, D), lambda i, ids: (0, 0)),
            pl.BlockSpec((GB, 1), lambda i, ids: (i, 0)),
            pl.BlockSpec((GB, D), lambda i, ids: (i, 0)),
            pl.BlockSpec((1, D), lambda i, ids: (0, 0)),
            pl.BlockSpec((D, D), lambda i, ids: (0, 0)),
            pl.BlockSpec((D, D), lambda i, ids: (0, 0)),
            pl.BlockSpec((1, D), lambda i, ids: (0, 0)),
            pl.BlockSpec((D, D), lambda i, ids: (0, 0)),
            pl.BlockSpec((1, D), lambda i, ids: (0, 0)),
        ],
        out_specs=pl.BlockSpec((GB, D), lambda i, ids: (i, 0)),
        scratch_shapes=[
            pltpu.VMEM((2, GB, N), jnp.float32),
            pltpu.VMEM((2, GB, D), jnp.float32),
            pltpu.SemaphoreType.DMA((2,)),
            pltpu.SemaphoreType.DMA((2,)),
        ],
    )
    return pl.pallas_call(
        _l2_body,
        grid_spec=grid_spec,
        out_shape=jax.ShapeDtypeStruct((B, D), jnp.float32),
    )(codeid, adj, init, s2, timediffs, features, patient_row, W_hh_T, W_f_T,
      w_t_row, W_p_T, b)


def kernel(patient_dynamic, code_dynamic, init_code_dynamic, adj, patientid,
           codeid, ancestorid, features, timediffs, W1, W2, W_ih, b_ih, W_hh,
           b_hh):
    s1 = _xw(code_dynamic, W1)
    x1 = _gcn_layer(adj, s1, init_code_dynamic)
    s2 = _xw(x1, W2)

    patient_row = jax.lax.dynamic_slice_in_dim(patient_dynamic, patientid, 1,
                                               axis=0)
    W_p_T = W_ih[:, :D].T
    w_t_row = W_ih[:, D:D + 1].T
    W_f_T = W_ih[:, D + 1:].T
    b = (b_ih + b_hh)[None, :]
    return _layer2_fused(codeid, adj, init_code_dynamic, s2, timediffs,
                         features, patient_row, W_hh.T, W_f_T, w_t_row, W_p_T,
                         b)


# speedup vs baseline: 1.4180x; 1.0200x over previous
import jax
import jax.numpy as jnp
from jax.experimental import pallas as pl
from jax.experimental.pallas import tpu as pltpu

N = 10000
D = 64
ALPHA = 0.5
ROWS_BLK = 400
NBUF = 3
GB = 256


def _xw_body(x_ref, w_ref, o_ref):
    o_ref[...] = jnp.dot(x_ref[...], w_ref[...], preferred_element_type=jnp.float32)


def _xw(x, w):
    return pl.pallas_call(
        _xw_body,
        out_shape=jax.ShapeDtypeStruct((x.shape[0], w.shape[1]), jnp.float32),
    )(x, w)


def _gcn_body(adj_ref, s_ref, init_ref, o_ref, a_buf, sem):
    i = pl.program_id(0)
    n = pl.num_programs(0)

    def issue(blk):
        pltpu.make_async_copy(
            adj_ref.at[pl.ds(blk * ROWS_BLK, ROWS_BLK), :],
            a_buf.at[blk % NBUF], sem.at[blk % NBUF]).start()

    @pl.when(i == 0)
    def _():
        issue(0)
        issue(1)

    @pl.when(i + 2 < n)
    def _():
        issue(i + 2)

    slot = i % NBUF
    pltpu.make_async_copy(
        adj_ref.at[pl.ds(i * ROWS_BLK, ROWS_BLK), :],
        a_buf.at[slot], sem.at[slot]).wait()
    y = jnp.dot(a_buf[slot], s_ref[...], preferred_element_type=jnp.float32)
    o_ref[...] = jnp.maximum(ALPHA * y + (1.0 - ALPHA) * init_ref[...], 0.0)


def _gcn_layer(adj, s, init):
    return pl.pallas_call(
        _gcn_body,
        grid=(N // ROWS_BLK,),
        in_specs=[
            pl.BlockSpec(memory_space=pltpu.MemorySpace.HBM),
            pl.BlockSpec((N, D), lambda i: (0, 0)),
            pl.BlockSpec((ROWS_BLK, D), lambda i: (i, 0)),
        ],
        out_specs=pl.BlockSpec((ROWS_BLK, D), lambda i: (i, 0)),
        out_shape=jax.ShapeDtypeStruct((N, D), jnp.float32),
        scratch_shapes=[
            pltpu.VMEM((NBUF, ROWS_BLK, N), jnp.float32),
            pltpu.SemaphoreType.DMA((NBUF,)),
        ],
    )(adj, s, init)


def _l2_body(id_ref, adj_ref, ig_ref, s_ref, td_ref, ft_ref, p_ref, whh_ref,
             wf_ref, wt_ref, wp_ref, b_ref, o_ref, a_buf, sem):
    i = pl.program_id(0)
    nblk = pl.num_programs(0)

    def issue(blk, slot):
        def body(j, _):
            r = id_ref[blk * GB + j]
            pltpu.make_async_copy(
                adj_ref.at[pl.ds(r, 1), :], a_buf.at[slot, pl.ds(j, 1), :],
                sem.at[slot]).start()
            return 0
        jax.lax.fori_loop(0, GB, body, 0, unroll=8)

    def wait(slot):
        def body(j, _):
            pltpu.make_async_copy(
                adj_ref.at[pl.ds(0, 1), :], a_buf.at[slot, pl.ds(0, 1), :],
                sem.at[slot]).wait()
            return 0
        jax.lax.fori_loop(0, GB, body, 0, unroll=8)

    @pl.when(i == 0)
    def _():
        issue(0, 0)

    @pl.when(i + 1 < nblk)
    def _():
        issue(i + 1, (i + 1) % 2)

    slot = i % 2
    wait(slot)
    y = jnp.dot(a_buf[slot], s_ref[...], preferred_element_type=jnp.float32)
    x2g = jnp.maximum(ALPHA * y + (1.0 - ALPHA) * ig_ref[...], 0.0)
    const = jnp.dot(p_ref[...], wp_ref[...], preferred_element_type=jnp.float32)
    z = jnp.dot(x2g, whh_ref[...], preferred_element_type=jnp.float32)
    z = z + jnp.dot(ft_ref[...], wf_ref[...], preferred_element_type=jnp.float32)
    z = z + td_ref[...] * wt_ref[...] + const + b_ref[...]
    h = jnp.tanh(z)
    nrm = jnp.sqrt(jnp.sum(h * h, axis=1, keepdims=True))
    o_ref[...] = h / jnp.maximum(nrm, 1e-12)


def _layer2_fused(codeid, adj, init_g, s2, timediffs, features, patient_row,
                  W_hh_T, W_f_T, w_t_row, W_p_T, b):
    B = codeid.shape[0]
    grid_spec = pltpu.PrefetchScalarGridSpec(
        num_scalar_prefetch=1,
        grid=(B // GB,),
        in_specs=[
            pl.BlockSpec(memory_space=pltpu.MemorySpace.HBM),
            pl.BlockSpec((GB, D), lambda i, ids: (i, 0)),
            pl.BlockSpec((N, D), lambda i, ids: (0, 0)),
            pl.BlockSpec((GB, 1), lambda i, ids: (i, 0)),
            pl.BlockSpec((GB, D), lambda i, ids: (i, 0)),
            pl.BlockSpec((1, D), lambda i, ids: (0, 0)),
            pl.BlockSpec((D, D), lambda i, ids: (0, 0)),
            pl.BlockSpec((D, D), lambda i, ids: (0, 0)),
            pl.BlockSpec((1, D), lambda i, ids: (0, 0)),
            pl.BlockSpec((D, D), lambda i, ids: (0, 0)),
            pl.BlockSpec((1, D), lambda i, ids: (0, 0)),
        ],
        out_specs=pl.BlockSpec((GB, D), lambda i, ids: (i, 0)),
        scratch_shapes=[
            pltpu.VMEM((2, GB, N), jnp.float32),
            pltpu.SemaphoreType.DMA((2,)),
        ],
    )
    return pl.pallas_call(
        _l2_body,
        grid_spec=grid_spec,
        out_shape=jax.ShapeDtypeStruct((B, D), jnp.float32),
    )(codeid, adj, init_g, s2, timediffs, features, patient_row, W_hh_T, W_f_T,
      w_t_row, W_p_T, b)


def kernel(patient_dynamic, code_dynamic, init_code_dynamic, adj, patientid,
           codeid, ancestorid, features, timediffs, W1, W2, W_ih, b_ih, W_hh,
           b_hh):
    s1 = _xw(code_dynamic, W1)
    x1 = _gcn_layer(adj, s1, init_code_dynamic)
    s2 = _xw(x1, W2)

    patient_row = jax.lax.dynamic_slice_in_dim(patient_dynamic, patientid, 1,
                                               axis=0)
    W_p_T = W_ih[:, :D].T
    w_t_row = W_ih[:, D:D + 1].T
    W_f_T = W_ih[:, D + 1:].T
    b = (b_ih + b_hh)[None, :]
    init_g = jnp.take(init_code_dynamic, codeid, axis=0)
    return _layer2_fused(codeid, adj, init_g, s2, timediffs,
                         features, patient_row, W_hh.T, W_f_T, w_t_row, W_p_T,
                         b)


# speedup vs baseline: 1.5667x; 1.1048x over previous
import jax
import jax.numpy as jnp
from jax.experimental import pallas as pl
from jax.experimental.pallas import tpu as pltpu

N = 10000
D = 64
ALPHA = 0.5
ROWS_BLK = 400
NBUF = 3
GB = 256
NBUF2 = 3


def _xw_body(x_ref, w_ref, o_ref):
    o_ref[...] = jnp.dot(x_ref[...], w_ref[...], preferred_element_type=jnp.float32)


def _xw(x, w):
    return pl.pallas_call(
        _xw_body,
        out_shape=jax.ShapeDtypeStruct((x.shape[0], w.shape[1]), jnp.float32),
    )(x, w)


def _gcn_body(adj_ref, s_ref, init_ref, o_ref, a_buf, sem):
    i = pl.program_id(0)
    n = pl.num_programs(0)

    def issue(blk):
        pltpu.make_async_copy(
            adj_ref.at[pl.ds(blk * ROWS_BLK, ROWS_BLK), :],
            a_buf.at[blk % NBUF], sem.at[blk % NBUF]).start()

    @pl.when(i == 0)
    def _():
        issue(0)
        issue(1)

    @pl.when(i + 2 < n)
    def _():
        issue(i + 2)

    slot = i % NBUF
    pltpu.make_async_copy(
        adj_ref.at[pl.ds(i * ROWS_BLK, ROWS_BLK), :],
        a_buf.at[slot], sem.at[slot]).wait()
    y = jnp.dot(a_buf[slot], s_ref[...], preferred_element_type=jnp.float32)
    o_ref[...] = jnp.maximum(ALPHA * y + (1.0 - ALPHA) * init_ref[...], 0.0)


def _gcn_layer(adj, s, init):
    return pl.pallas_call(
        _gcn_body,
        grid=(N // ROWS_BLK,),
        in_specs=[
            pl.BlockSpec(memory_space=pltpu.MemorySpace.HBM),
            pl.BlockSpec((N, D), lambda i: (0, 0)),
            pl.BlockSpec((ROWS_BLK, D), lambda i: (i, 0)),
        ],
        out_specs=pl.BlockSpec((ROWS_BLK, D), lambda i: (i, 0)),
        out_shape=jax.ShapeDtypeStruct((N, D), jnp.float32),
        scratch_shapes=[
            pltpu.VMEM((NBUF, ROWS_BLK, N), jnp.float32),
            pltpu.SemaphoreType.DMA((NBUF,)),
        ],
    )(adj, s, init)


def _l2_body(id_ref, adj_ref, init_ref, s_ref, td_ref, ft_ref, p_ref, whh_ref,
             wf_ref, wt_ref, wp_ref, b_ref, o_ref, a_buf, i_buf, sem, isem):
    i = pl.program_id(0)
    nblk = pl.num_programs(0)

    def issue(blk):
        slot = blk % NBUF2

        def body(j, _):
            r = id_ref[blk * GB + j]
            pltpu.make_async_copy(
                adj_ref.at[pl.ds(r, 1), :], a_buf.at[slot, pl.ds(j, 1), :],
                sem.at[slot]).start()
            pltpu.make_async_copy(
                init_ref.at[pl.ds(r, 1), :], i_buf.at[slot, pl.ds(j, 1), :],
                isem.at[slot]).start()
            return 0
        jax.lax.fori_loop(0, GB, body, 0, unroll=8)

    def wait(slot):
        def body(j, _):
            pltpu.make_async_copy(
                adj_ref.at[pl.ds(0, 1), :], a_buf.at[slot, pl.ds(0, 1), :],
                sem.at[slot]).wait()
            pltpu.make_async_copy(
                init_ref.at[pl.ds(0, 1), :], i_buf.at[slot, pl.ds(0, 1), :],
                isem.at[slot]).wait()
            return 0
        jax.lax.fori_loop(0, GB, body, 0, unroll=8)

    @pl.when(i == 0)
    def _():
        issue(0)
        issue(1)

    @pl.when(i + 2 < nblk)
    def _():
        issue(i + 2)

    slot = i % NBUF2
    wait(slot)
    y = jnp.dot(a_buf[slot], s_ref[...], preferred_element_type=jnp.float32)
    x2g = jnp.maximum(ALPHA * y + (1.0 - ALPHA) * i_buf[slot], 0.0)
    const = jnp.dot(p_ref[...], wp_ref[...], preferred_element_type=jnp.float32)
    z = jnp.dot(x2g, whh_ref[...], preferred_element_type=jnp.float32)
    z = z + jnp.dot(ft_ref[...], wf_ref[...], preferred_element_type=jnp.float32)
    z = z + td_ref[...] * wt_ref[...] + const + b_ref[...]
    h = jnp.tanh(z)
    nrm = jnp.sqrt(jnp.sum(h * h, axis=1, keepdims=True))
    o_ref[...] = h / jnp.maximum(nrm, 1e-12)


def _layer2_fused(codeid, adj, init, s2, timediffs, features, patient_row,
                  W_hh_T, W_f_T, w_t_row, W_p_T, b):
    B = codeid.shape[0]
    grid_spec = pltpu.PrefetchScalarGridSpec(
        num_scalar_prefetch=1,
        grid=(B // GB,),
        in_specs=[
            pl.BlockSpec(memory_space=pltpu.MemorySpace.HBM),
            pl.BlockSpec(memory_space=pltpu.MemorySpace.HBM),
            pl.BlockSpec((N, D), lambda i, ids: (0, 0)),
            pl.BlockSpec((GB, 1), lambda i, ids: (i, 0)),
            pl.BlockSpec((GB, D), lambda i, ids: (i, 0)),
            pl.BlockSpec((1, D), lambda i, ids: (0, 0)),
            pl.BlockSpec((D, D), lambda i, ids: (0, 0)),
            pl.BlockSpec((D, D), lambda i, ids: (0, 0)),
            pl.BlockSpec((1, D), lambda i, ids: (0, 0)),
            pl.BlockSpec((D, D), lambda i, ids: (0, 0)),
            pl.BlockSpec((1, D), lambda i, ids: (0, 0)),
        ],
        out_specs=pl.BlockSpec((GB, D), lambda i, ids: (i, 0)),
        scratch_shapes=[
            pltpu.VMEM((NBUF2, GB, N), jnp.float32),
            pltpu.VMEM((NBUF2, GB, D), jnp.float32),
            pltpu.SemaphoreType.DMA((NBUF2,)),
            pltpu.SemaphoreType.DMA((NBUF2,)),
        ],
    )
    return pl.pallas_call(
        _l2_body,
        grid_spec=grid_spec,
        out_shape=jax.ShapeDtypeStruct((B, D), jnp.float32),
    )(codeid, adj, init, s2, timediffs, features, patient_row, W_hh_T, W_f_T,
      w_t_row, W_p_T, b)


def kernel(patient_dynamic, code_dynamic, init_code_dynamic, adj, patientid,
           codeid, ancestorid, features, timediffs, W1, W2, W_ih, b_ih, W_hh,
           b_hh):
    s1 = _xw(code_dynamic, W1)
    x1 = _gcn_layer(adj, s1, init_code_dynamic)
    s2 = _xw(x1, W2)

    patient_row = jax.lax.dynamic_slice_in_dim(patient_dynamic, patientid, 1,
                                               axis=0)
    W_p_T = W_ih[:, :D].T
    w_t_row = W_ih[:, D:D + 1].T
    W_f_T = W_ih[:, D + 1:].T
    b = (b_ih + b_hh)[None, :]
    return _layer2_fused(codeid, adj, init_code_dynamic, s2, timediffs,
                         features, patient_row, W_hh.T, W_f_T, w_t_row, W_p_T,
                         b)


# speedup vs baseline: 1.6275x; 1.0388x over previous
import jax
import jax.numpy as jnp
from jax.experimental import pallas as pl
from jax.experimental.pallas import tpu as pltpu

N = 10000
D = 64
B = 2048
ALPHA = 0.5
RB = 400
NA = N // RB
GB = 256
NB = B // GB
NBUF = 3


def _body(id_ref, adj_ref, init_ref, x0_ref, iga_ref, w1_ref, w2_ref, td_ref,
          ft_ref, p_ref, whh_ref, wf_ref, wt_ref, wp_ref, b_ref, o_ref,
          a_buf, i_buf, s_buf, sem, isem):
    i = pl.program_id(0)
    nsteps = pl.num_programs(0)

    def issue(blk):
        slot = blk % NBUF

        @pl.when(blk < NA)
        def _():
            pltpu.make_async_copy(
                adj_ref.at[pl.ds(blk * RB, RB), :], a_buf.at[slot],
                sem.at[slot]).start()

        @pl.when(blk >= NA)
        def _():
            def body(j, _):
                r = id_ref[(blk - NA) * GB + j]
                pltpu.make_async_copy(
                    adj_ref.at[pl.ds(r, 1), :],
                    a_buf.at[slot, pl.ds(j, 1), :], sem.at[slot]).start()
                pltpu.make_async_copy(
                    init_ref.at[pl.ds(r, 1), :],
                    i_buf.at[slot, pl.ds(j, 1), :], isem.at[slot]).start()
                return 0
            jax.lax.fori_loop(0, GB, body, 0, unroll=8)

    @pl.when(i == 0)
    def _():
        issue(0)
        issue(1)

    @pl.when(i + 2 < nsteps)
    def _():
        issue(i + 2)

    slot = i % NBUF

    @pl.when(i < NA)
    def _():
        pltpu.make_async_copy(
            adj_ref.at[pl.ds(0, RB), :], a_buf.at[slot], sem.at[slot]).wait()

        @pl.when(i == 0)
        def _():
            s_buf[:, :D] = jnp.dot(x0_ref[...], w1_ref[...],
                                   preferred_element_type=jnp.float32)

        y = jnp.dot(a_buf[slot], s_buf[:, :D],
                    preferred_element_type=jnp.float32)
        x1b = jnp.maximum(ALPHA * y + (1.0 - ALPHA) * iga_ref[...], 0.0)
        s_buf[pl.ds(i * RB, RB), D:] = jnp.dot(
            x1b, w2_ref[...], preferred_element_type=jnp.float32)

    @pl.when(i >= NA)
    def _():
        def wbody(j, _):
            pltpu.make_async_copy(
                adj_ref.at[pl.ds(0, 1), :], a_buf.at[slot, pl.ds(0, 1), :],
                sem.at[slot]).wait()
            pltpu.make_async_copy(
                init_ref.at[pl.ds(0, 1), :], i_buf.at[slot, pl.ds(0, 1), :],
                isem.at[slot]).wait()
            return 0
        jax.lax.fori_loop(0, GB, wbody, 0, unroll=8)

        y = jnp.dot(a_buf[slot, :GB, :], s_buf[:, D:],
                    preferred_element_type=jnp.float32)
        x2g = jnp.maximum(ALPHA * y + (1.0 - ALPHA) * i_buf[slot], 0.0)
        const = jnp.dot(p_ref[...], wp_ref[...],
                        preferred_element_type=jnp.float32)
        z = jnp.dot(x2g, whh_ref[...], preferred_element_type=jnp.float32)
        z = z + jnp.dot(ft_ref[...], wf_ref[...],
                        preferred_element_type=jnp.float32)
        z = z + td_ref[...] * wt_ref[...] + const + b_ref[...]
        h = jnp.tanh(z)
        nrm = jnp.sqrt(jnp.sum(h * h, axis=1, keepdims=True))
        o_ref[...] = h / jnp.maximum(nrm, 1e-12)


def kernel(patient_dynamic, code_dynamic, init_code_dynamic, adj, patientid,
           codeid, ancestorid, features, timediffs, W1, W2, W_ih, b_ih, W_hh,
           b_hh):
    patient_row = jax.lax.dynamic_slice_in_dim(patient_dynamic, patientid, 1,
                                               axis=0)
    W_p_T = W_ih[:, :D].T
    w_t_row = W_ih[:, D:D + 1].T
    W_f_T = W_ih[:, D + 1:].T
    b = (b_ih + b_hh)[None, :]

    grid_spec = pltpu.PrefetchScalarGridSpec(
        num_scalar_prefetch=1,
        grid=(NA + NB,),
        in_specs=[
            pl.BlockSpec(memory_space=pltpu.MemorySpace.HBM),
            pl.BlockSpec(memory_space=pltpu.MemorySpace.HBM),
            pl.BlockSpec((N, D), lambda i, ids: (0, 0)),
            pl.BlockSpec((RB, D),
                         lambda i, ids: (jnp.minimum(i, NA - 1), 0)),
            pl.BlockSpec((D, D), lambda i, ids: (0, 0)),
            pl.BlockSpec((D, D), lambda i, ids: (0, 0)),
            pl.BlockSpec((GB, 1),
                         lambda i, ids: (jnp.maximum(i - NA, 0), 0)),
            pl.BlockSpec((GB, D),
                         lambda i, ids: (jnp.maximum(i - NA, 0), 0)),
            pl.BlockSpec((1, D), lambda i, ids: (0, 0)),
            pl.BlockSpec((D, D), lambda i, ids: (0, 0)),
            pl.BlockSpec((D, D), lambda i, ids: (0, 0)),
            pl.BlockSpec((1, D), lambda i, ids: (0, 0)),
            pl.BlockSpec((D, D), lambda i, ids: (0, 0)),
            pl.BlockSpec((1, D), lambda i, ids: (0, 0)),
        ],
        out_specs=pl.BlockSpec((GB, D),
                               lambda i, ids: (jnp.maximum(i - NA, 0), 0)),
        scratch_shapes=[
            pltpu.VMEM((NBUF, RB, N), jnp.float32),
            pltpu.VMEM((NBUF, GB, D), jnp.float32),
            pltpu.VMEM((N, 2 * D), jnp.float32),
            pltpu.SemaphoreType.DMA((NBUF,)),
            pltpu.SemaphoreType.DMA((NBUF,)),
        ],
    )
    return pl.pallas_call(
        _body,
        grid_spec=grid_spec,
        out_shape=jax.ShapeDtypeStruct((B, D), jnp.float32),
    )(codeid, adj, init_code_dynamic, code_dynamic, init_code_dynamic, W1, W2,
      timediffs, features, patient_row, W_hh.T, W_f_T, w_t_row, W_p_T, b)


# speedup vs baseline: 1.6361x; 1.0053x over previous
import jax
import jax.numpy as jnp
from jax.experimental import pallas as pl
from jax.experimental.pallas import tpu as pltpu

N = 10000
D = 64
B = 2048
ALPHA = 0.5
RB = 400
NA = N // RB
GB = 256
NB = B // GB
NBUF = 3


def _body(id_ref, adj_ref, init_ref, x0_ref, iga_ref, w1_ref, w2_ref, td_ref,
          ft_ref, p_ref, whh_ref, wf_ref, wt_ref, wp_ref, b_ref, o_ref,
          a_buf, i_buf, s_buf, sem, isem):
    i = pl.program_id(0)
    nsteps = pl.num_programs(0)

    def issue(blk):
        slot = blk % NBUF

        @pl.when(blk < NA)
        def _():
            h = RB // 2
            pltpu.make_async_copy(
                adj_ref.at[pl.ds(blk * RB, h), :],
                a_buf.at[slot, pl.ds(0, h), :], sem.at[slot]).start()
            pltpu.make_async_copy(
                adj_ref.at[pl.ds(blk * RB + h, h), :],
                a_buf.at[slot, pl.ds(h, h), :], isem.at[slot]).start()

        @pl.when(blk >= NA)
        def _():
            def body(j, _):
                r = id_ref[(blk - NA) * GB + j]
                pltpu.make_async_copy(
                    adj_ref.at[pl.ds(r, 1), :],
                    a_buf.at[slot, pl.ds(j, 1), :], sem.at[slot]).start()
                pltpu.make_async_copy(
                    init_ref.at[pl.ds(r, 1), :],
                    i_buf.at[slot, pl.ds(j, 1), :], isem.at[slot]).start()
                return 0
            jax.lax.fori_loop(0, GB, body, 0, unroll=8)

    @pl.when(i == 0)
    def _():
        issue(0)
        issue(1)

    @pl.when(i + 2 < nsteps)
    def _():
        issue(i + 2)

    slot = i % NBUF

    @pl.when(i < NA)
    def _():
        h = RB // 2
        pltpu.make_async_copy(
            adj_ref.at[pl.ds(0, h), :], a_buf.at[slot, pl.ds(0, h), :],
            sem.at[slot]).wait()
        pltpu.make_async_copy(
            adj_ref.at[pl.ds(0, h), :], a_buf.at[slot, pl.ds(0, h), :],
            isem.at[slot]).wait()

        @pl.when(i == 0)
        def _():
            s_buf[:, :D] = jnp.dot(x0_ref[...], w1_ref[...],
                                   preferred_element_type=jnp.float32)

        y = jnp.dot(a_buf[slot], s_buf[:, :D],
                    preferred_element_type=jnp.float32)
        x1b = jnp.maximum(ALPHA * y + (1.0 - ALPHA) * iga_ref[...], 0.0)
        s_buf[pl.ds(i * RB, RB), D:] = jnp.dot(
            x1b, w2_ref[...], preferred_element_type=jnp.float32)

    @pl.when(i >= NA)
    def _():
        def wbody(j, _):
            pltpu.make_async_copy(
                adj_ref.at[pl.ds(0, 1), :], a_buf.at[slot, pl.ds(0, 1), :],
                sem.at[slot]).wait()
            pltpu.make_async_copy(
                init_ref.at[pl.ds(0, 1), :], i_buf.at[slot, pl.ds(0, 1), :],
                isem.at[slot]).wait()
            return 0
        jax.lax.fori_loop(0, GB, wbody, 0, unroll=8)

        y = jnp.dot(a_buf[slot, :GB, :], s_buf[:, D:],
                    preferred_element_type=jnp.float32)
        x2g = jnp.maximum(ALPHA * y + (1.0 - ALPHA) * i_buf[slot], 0.0)
        const = jnp.dot(p_ref[...], wp_ref[...],
                        preferred_element_type=jnp.float32)
        z = jnp.dot(x2g, whh_ref[...], preferred_element_type=jnp.float32)
        z = z + jnp.dot(ft_ref[...], wf_ref[...],
                        preferred_element_type=jnp.float32)
        z = z + td_ref[...] * wt_ref[...] + const + b_ref[...]
        h = jnp.tanh(z)
        nrm = jnp.sqrt(jnp.sum(h * h, axis=1, keepdims=True))
        o_ref[...] = h / jnp.maximum(nrm, 1e-12)


def kernel(patient_dynamic, code_dynamic, init_code_dynamic, adj, patientid,
           codeid, ancestorid, features, timediffs, W1, W2, W_ih, b_ih, W_hh,
           b_hh):
    patient_row = jax.lax.dynamic_slice_in_dim(patient_dynamic, patientid, 1,
                                               axis=0)
    W_p_T = W_ih[:, :D].T
    w_t_row = W_ih[:, D:D + 1].T
    W_f_T = W_ih[:, D + 1:].T
    b = (b_ih + b_hh)[None, :]

    grid_spec = pltpu.PrefetchScalarGridSpec(
        num_scalar_prefetch=1,
        grid=(NA + NB,),
        in_specs=[
            pl.BlockSpec(memory_space=pltpu.MemorySpace.HBM),
            pl.BlockSpec(memory_space=pltpu.MemorySpace.HBM),
            pl.BlockSpec((N, D), lambda i, ids: (0, 0)),
            pl.BlockSpec((RB, D),
                         lambda i, ids: (jnp.minimum(i, NA - 1), 0)),
            pl.BlockSpec((D, D), lambda i, ids: (0, 0)),
            pl.BlockSpec((D, D), lambda i, ids: (0, 0)),
            pl.BlockSpec((GB, 1),
                         lambda i, ids: (jnp.maximum(i - NA, 0), 0)),
            pl.BlockSpec((GB, D),
                         lambda i, ids: (jnp.maximum(i - NA, 0), 0)),
            pl.BlockSpec((1, D), lambda i, ids: (0, 0)),
            pl.BlockSpec((D, D), lambda i, ids: (0, 0)),
            pl.BlockSpec((D, D), lambda i, ids: (0, 0)),
            pl.BlockSpec((1, D), lambda i, ids: (0, 0)),
            pl.BlockSpec((D, D), lambda i, ids: (0, 0)),
            pl.BlockSpec((1, D), lambda i, ids: (0, 0)),
        ],
        out_specs=pl.BlockSpec((GB, D),
                               lambda i, ids: (jnp.maximum(i - NA, 0), 0)),
        scratch_shapes=[
            pltpu.VMEM((NBUF, RB, N), jnp.float32),
            pltpu.VMEM((NBUF, GB, D), jnp.float32),
            pltpu.VMEM((N, 2 * D), jnp.float32),
            pltpu.SemaphoreType.DMA((NBUF,)),
            pltpu.SemaphoreType.DMA((NBUF,)),
        ],
    )
    return pl.pallas_call(
        _body,
        grid_spec=grid_spec,
        out_shape=jax.ShapeDtypeStruct((B, D), jnp.float32),
    )(codeid, adj, init_code_dynamic, code_dynamic, init_code_dynamic, W1, W2,
      timediffs, features, patient_row, W_hh.T, W_f_T, w_t_row, W_p_T, b)
